# Initial kernel scaffold; baseline (speedup 1.0000x reference)
import jax
import jax.numpy as jnp
from jax.experimental import pallas as pl


def kernel(x, gate_w, gate_b, w1, b1, w2, b2):
    raise NotImplementedError("write your pallas kernel here")


# speedup vs baseline: 1.7281x; 1.7281x over previous
import functools

import jax
import jax.numpy as jnp
from jax.experimental import pallas as pl
from jax.experimental.pallas import tpu as pltpu

E = 8
K = 2
T = 2048
D = 768
F = 768
LANES = 128
NEG = -1e30


def _moe_dense_body(x_ref, gwp_ref, gbp_ref, w1_ref, b1_ref, w2_ref, b2_ref,
                    out_ref, c1_ref, c2_ref, i1_ref, i2_ref):
    e = pl.program_id(0)

    @pl.when(e == 0)
    def _gate():
        xb = x_ref[...]
        logits = jnp.dot(xb, gwp_ref[...], preferred_element_type=jnp.float32)
        logits = logits + gbp_ref[...]
        m = jnp.max(logits, axis=-1, keepdims=True)
        p = jnp.exp(logits - m)
        gates = p / jnp.sum(p, axis=-1, keepdims=True)
        iota = jax.lax.broadcasted_iota(jnp.int32, (T, LANES), 1)
        v1 = jnp.max(gates, axis=-1, keepdims=True)
        i1 = jnp.min(jnp.where(gates == v1, iota, LANES), axis=-1, keepdims=True)
        g2 = jnp.where(iota == i1, NEG, gates)
        v2 = jnp.max(g2, axis=-1, keepdims=True)
        i2 = jnp.min(jnp.where(g2 == v2, iota, LANES), axis=-1, keepdims=True)
        s = v1 + v2
        c1_ref[...] = v1 / s
        c2_ref[...] = v2 / s
        i1_ref[...] = i1
        i2_ref[...] = i2

    xb = x_ref[...]
    h = jnp.dot(xb, w1_ref[0], preferred_element_type=jnp.float32) + b1_ref[0]
    h = jnp.maximum(h, 0.0)
    y = jnp.dot(h, w2_ref[0], preferred_element_type=jnp.float32) + b2_ref[0]
    w_e = (jnp.where(i1_ref[...] == e, c1_ref[...], 0.0)
           + jnp.where(i2_ref[...] == e, c2_ref[...], 0.0))

    @pl.when(e == 0)
    def _init():
        out_ref[...] = y * w_e

    @pl.when(e != 0)
    def _acc():
        out_ref[...] += y * w_e


@jax.jit
def kernel(x, gate_w, gate_b, w1, b1, w2, b2):
    gwp = jnp.zeros((D, LANES), jnp.float32).at[:, :E].set(gate_w)
    gbp = jnp.full((LANES,), NEG, jnp.float32).at[:E].set(gate_b)
    b1r = b1.reshape(E, 1, F)
    b2r = b2.reshape(E, 1, D)

    grid = (E,)
    out = pl.pallas_call(
        _moe_dense_body,
        grid=grid,
        in_specs=[
            pl.BlockSpec((T, D), lambda e: (0, 0)),
            pl.BlockSpec((D, LANES), lambda e: (0, 0)),
            pl.BlockSpec((LANES,), lambda e: (0,)),
            pl.BlockSpec((1, D, F), lambda e: (e, 0, 0)),
            pl.BlockSpec((1, 1, F), lambda e: (e, 0, 0)),
            pl.BlockSpec((1, F, D), lambda e: (e, 0, 0)),
            pl.BlockSpec((1, 1, D), lambda e: (e, 0, 0)),
        ],
        out_specs=pl.BlockSpec((T, D), lambda e: (0, 0)),
        out_shape=jax.ShapeDtypeStruct((T, D), jnp.float32),
        scratch_shapes=[
            pltpu.VMEM((T, 1), jnp.float32),
            pltpu.VMEM((T, 1), jnp.float32),
            pltpu.VMEM((T, 1), jnp.int32),
            pltpu.VMEM((T, 1), jnp.int32),
        ],
    )(x, gwp, gbp, w1, b1r, w2, b2r)
    return out
